# Initial kernel scaffold; baseline (speedup 1.0000x reference)
import jax
import jax.numpy as jnp
from jax.experimental import pallas as pl


def kernel(target_x, target_edge_index, e3_ligase_x, e3_ligase_edge_index, protac_x, protac_edge_index, W1t, b1t, W2t, b2t, W1e, b1e, W2e, b2e, W1p, b1p, W2p, b2p, Wfc, bfc):
    raise NotImplementedError("write your pallas kernel here")


# speedup vs baseline: 77.9087x; 77.9087x over previous
import functools

import jax
import jax.numpy as jnp
from jax import lax
from jax.experimental import pallas as pl
from jax.experimental.pallas import tpu as pltpu
from jax.experimental.pallas import tpu_sc as plsc

N = 100000
E = 3200000
LANES = 128
R = E // LANES
K = 8
CHUNKS = R // K
NC, NS = 2, 16
NW = NC * NS
NPAD = 100352
STRIPE = NPAD // NS
BLK = 2048
NB = NPAD // BLK


def _deg_body(e_t, e_e, e_p, ones_hbm, z1_hbm, out_t, out_e, out_p,
              deg_t, deg_e, deg_p, ones_v, idx_v, stage1, sem):
    cid = lax.axis_index("c")
    sid = lax.axis_index("s")
    wid = cid * NS + sid
    off = sid * STRIPE
    pltpu.sync_copy(ones_hbm, ones_v)
    pltpu.sync_copy(z1_hbm, stage1)
    for deg in (deg_t, deg_e, deg_p):
        pltpu.sync_copy(stage1, deg.at[pl.ds(off, STRIPE)])
    plsc.subcore_barrier()
    for e_ref, deg in ((e_t, deg_t), (e_e, deg_e), (e_p, deg_p)):
        n_chunks = (CHUNKS - wid + NW - 1) // NW

        def body(i, carry, e_ref=e_ref, deg=deg):
            row0 = (wid + NW * i) * K
            pltpu.sync_copy(e_ref.at[1, pl.ds(row0, K)], idx_v)
            descs = [
                pltpu.async_copy(ones_v, deg.at[idx_v.at[j]], sem, add=True)
                for j in range(K)
            ]
            for d in descs:
                d.wait()
            return carry

        lax.fori_loop(0, n_chunks, body, 0)
    plsc.subcore_barrier()
    for deg, out in ((deg_t, out_t), (deg_e, out_e), (deg_p, out_p)):
        pltpu.sync_copy(deg.at[pl.ds(off, STRIPE)], stage1)
        pltpu.sync_copy(stage1, out.at[pl.ds(cid * NPAD + off, STRIPE)])


def _msg_body(e_t, e_e, e_p, g4_t, g4_e, g4_p, dv_t, dv_e, dv_p,
              z4_hbm, z1_hbm, ao_t, ao_e, ao_p, so_t, so_e, so_p,
              agg, s, idx_src, idx_dst, rows, dvals, stage4, stage1, sem):
    cid = lax.axis_index("c")
    sid = lax.axis_index("s")
    wid = cid * NS + sid
    off = sid * STRIPE
    for e_ref, g4, dv, ao, so in (
        (e_t, g4_t, dv_t, ao_t, so_t),
        (e_e, g4_e, dv_e, ao_e, so_e),
        (e_p, g4_p, dv_p, ao_p, so_p),
    ):
        pltpu.sync_copy(z4_hbm, stage4)
        pltpu.sync_copy(stage4, agg.at[pl.ds(off, STRIPE), :])
        pltpu.sync_copy(z1_hbm, stage1)
        pltpu.sync_copy(stage1, s.at[pl.ds(off, STRIPE)])
        plsc.subcore_barrier()
        n_chunks = (CHUNKS - wid + NW - 1) // NW

        def body(i, carry, e_ref=e_ref, g4=g4, dv=dv):
            row0 = (wid + NW * i) * K
            pltpu.sync_copy(e_ref.at[0, pl.ds(row0, K)], idx_src)
            pltpu.sync_copy(e_ref.at[1, pl.ds(row0, K)], idx_dst)
            descs = [
                pltpu.async_copy(g4.at[idx_src.at[j]], rows.at[j], sem)
                for j in range(K)
            ] + [
                pltpu.async_copy(dv.at[idx_dst.at[j]], dvals.at[j], sem)
                for j in range(K)
            ]
            for d in descs:
                d.wait()
            descs = [
                pltpu.async_copy(rows.at[j], agg.at[idx_dst.at[j]], sem, add=True)
                for j in range(K)
            ] + [
                pltpu.async_copy(dvals.at[j], s.at[idx_src.at[j]], sem, add=True)
                for j in range(K)
            ]
            for d in descs:
                d.wait()
            return carry

        lax.fori_loop(0, n_chunks, body, 0)
        plsc.subcore_barrier()
        pltpu.sync_copy(agg.at[pl.ds(off, STRIPE), :], stage4)
        pltpu.sync_copy(stage4, ao.at[pl.ds(cid * NPAD + off, STRIPE), :])
        pltpu.sync_copy(s.at[pl.ds(off, STRIPE)], stage1)
        pltpu.sync_copy(stage1, so.at[pl.ds(cid * NPAD + off, STRIPE)])


def _prep_body(degp_ref, x_ref, dinv_ref, g4_ref):
    i = pl.program_id(0)
    dsum = degp_ref[:, 0, :] + degp_ref[:, 1, :] + 1.0
    rowid = lax.broadcasted_iota(jnp.int32, (3, BLK), 1) + i * BLK
    dinv = jnp.where(rowid < N, lax.rsqrt(dsum), 0.0)
    dinv_ref[...] = dinv
    g4_ref[...] = x_ref[...] * dinv[:, :, None]


def _acc_body(aggp_ref, sp_ref, dinv_ref, g4_ref, w1_ref, b1_ref, out_ref):
    i = pl.program_id(0)
    a4 = aggp_ref[:, 0] + aggp_ref[:, 1] + g4_ref[...]
    dinv = dinv_ref[...]
    s2 = sp_ref[:, 0, :] + sp_ref[:, 1, :]
    crow = dinv * (s2 + dinv)
    accs = []
    for g in range(3):
        zw = jnp.dot(a4[g], w1_ref[g], preferred_element_type=jnp.float32)
        dinv_col = jnp.transpose(dinv[g:g + 1, :])
        h = jax.nn.relu(zw * dinv_col + b1_ref[g:g + 1, :])
        accs.append(jnp.dot(crow[g:g + 1, :], h,
                            preferred_element_type=jnp.float32))

    @pl.when(i == 0)
    def _():
        out_ref[...] = jnp.zeros_like(out_ref)

    out_ref[...] += jnp.concatenate(accs, axis=0)


def _head_body(accs_ref, w2_ref, b2_ref, wfc_ref, bfc_ref, out_ref):
    ms = []
    for g in range(3):
        a = accs_ref[g:g + 1, :]
        m = jnp.dot(a, w2_ref[g], preferred_element_type=jnp.float32)
        ms.append(m / float(N) + b2_ref[g:g + 1, :])
    comb = jnp.concatenate(ms, axis=1)
    o = jnp.dot(comb, wfc_ref[...], preferred_element_type=jnp.float32)
    out_ref[...] = jax.nn.sigmoid(o + bfc_ref[...])


def _pad_x(x):
    return jnp.pad(x, ((0, NPAD - N), (0, 0)))


def kernel(target_x, target_edge_index, e3_ligase_x, e3_ligase_edge_index,
           protac_x, protac_edge_index, W1t, b1t, W2t, b2t, W1e, b1e, W2e, b2e,
           W1p, b1p, W2p, b2p, Wfc, bfc):
    f32 = jnp.float32
    e_t = jnp.reshape(target_edge_index.astype(jnp.int32), (2, R, LANES))
    e_e = jnp.reshape(e3_ligase_edge_index.astype(jnp.int32), (2, R, LANES))
    e_p = jnp.reshape(protac_edge_index.astype(jnp.int32), (2, R, LANES))
    xs = jnp.stack([_pad_x(target_x), _pad_x(e3_ligase_x), _pad_x(protac_x)])

    ones_hbm = jnp.ones((LANES,), f32)
    z1_hbm = jnp.zeros((STRIPE,), f32)
    z4_hbm = jnp.zeros((STRIPE, 4), f32)

    mesh = plsc.VectorSubcoreMesh(
        core_axis_name="c", subcore_axis_name="s",
        num_cores=NC, num_subcores=NS)

    degp = pl.kernel(
        _deg_body,
        compiler_params=pltpu.CompilerParams(use_tc_tiling_on_sc=False),
        out_type=[jax.ShapeDtypeStruct((NC * NPAD,), f32)] * 3,
        mesh=mesh,
        scratch_types=[
            pltpu.VMEM_SHARED((NPAD,), f32),
            pltpu.VMEM_SHARED((NPAD,), f32),
            pltpu.VMEM_SHARED((NPAD,), f32),
            pltpu.VMEM((LANES,), f32),
            pltpu.VMEM((K, LANES), jnp.int32),
            pltpu.VMEM((STRIPE,), f32),
            pltpu.SemaphoreType.DMA,
        ],
    )(e_t, e_e, e_p, ones_hbm, z1_hbm)
    degp = jnp.stack(degp).reshape(3, NC, NPAD)

    dinv_all, g4_all = pl.pallas_call(
        _prep_body,
        grid=(NB,),
        in_specs=[
            pl.BlockSpec((3, NC, BLK), lambda i: (0, 0, i)),
            pl.BlockSpec((3, BLK, 4), lambda i: (0, i, 0)),
        ],
        out_specs=[
            pl.BlockSpec((3, BLK), lambda i: (0, i)),
            pl.BlockSpec((3, BLK, 4), lambda i: (0, i, 0)),
        ],
        out_shape=[
            jax.ShapeDtypeStruct((3, NPAD), f32),
            jax.ShapeDtypeStruct((3, NPAD, 4), f32),
        ],
    )(degp, xs)

    msg_outs = pl.kernel(
        _msg_body,
        compiler_params=pltpu.CompilerParams(use_tc_tiling_on_sc=False),
        out_type=[jax.ShapeDtypeStruct((NC * NPAD, 4), f32)] * 3
                 + [jax.ShapeDtypeStruct((NC * NPAD,), f32)] * 3,
        mesh=mesh,
        scratch_types=[
            pltpu.VMEM_SHARED((NPAD, 4), f32),
            pltpu.VMEM_SHARED((NPAD,), f32),
            pltpu.VMEM((K, LANES), jnp.int32),
            pltpu.VMEM((K, LANES), jnp.int32),
            pltpu.VMEM((K, LANES, 4), f32),
            pltpu.VMEM((K, LANES), f32),
            pltpu.VMEM((STRIPE, 4), f32),
            pltpu.VMEM((STRIPE,), f32),
            pltpu.SemaphoreType.DMA,
        ],
    )(e_t, e_e, e_p,
      g4_all[0], g4_all[1], g4_all[2],
      dinv_all[0], dinv_all[1], dinv_all[2],
      z4_hbm, z1_hbm)
    aggp = jnp.stack(msg_outs[:3]).reshape(3, NC, NPAD, 4)
    sp = jnp.stack(msg_outs[3:]).reshape(3, NC, NPAD)

    w1s = jnp.stack([W1t, W1e, W1p])
    b1s = jnp.stack([b1t, b1e, b1p])
    accs = pl.pallas_call(
        _acc_body,
        grid=(NB,),
        in_specs=[
            pl.BlockSpec((3, NC, BLK, 4), lambda i: (0, 0, i, 0)),
            pl.BlockSpec((3, NC, BLK), lambda i: (0, 0, i)),
            pl.BlockSpec((3, BLK), lambda i: (0, i)),
            pl.BlockSpec((3, BLK, 4), lambda i: (0, i, 0)),
            pl.BlockSpec((3, 4, 64), lambda i: (0, 0, 0)),
            pl.BlockSpec((3, 64), lambda i: (0, 0)),
        ],
        out_specs=pl.BlockSpec((3, 64), lambda i: (0, 0)),
        out_shape=jax.ShapeDtypeStruct((3, 64), f32),
    )(aggp, sp, dinv_all, g4_all, w1s, b1s)

    w2s = jnp.stack([W2t, W2e, W2p])
    b2s = jnp.stack([b2t, b2e, b2p])
    out = pl.pallas_call(
        _head_body,
        out_shape=jax.ShapeDtypeStruct((1, 1), f32),
    )(accs, w2s, b2s, Wfc, bfc.reshape(1, 1))
    return out.reshape(1)


# speedup vs baseline: 94.8747x; 1.2178x over previous
import functools

import jax
import jax.numpy as jnp
from jax import lax
from jax.experimental import pallas as pl
from jax.experimental.pallas import tpu as pltpu
from jax.experimental.pallas import tpu_sc as plsc

N = 100000
E = 3200000
LANES = 128
R = E // LANES
K = 25
CHUNKS = R // K
NC, NS = 2, 16
NW = NC * NS
NPAD = 100352
STRIPE = NPAD // NS
BLK = 2048
NB = NPAD // BLK


def _deg_body(e_t, e_e, e_p, ones_hbm, z1_hbm, out_t, out_e, out_p,
              deg_t, deg_e, deg_p, ones_v, idx_v, stage1, sem):
    cid = lax.axis_index("c")
    sid = lax.axis_index("s")
    wid = cid * NS + sid
    off = sid * STRIPE
    pltpu.sync_copy(ones_hbm, ones_v)
    pltpu.sync_copy(z1_hbm, stage1)
    for deg in (deg_t, deg_e, deg_p):
        pltpu.sync_copy(stage1, deg.at[pl.ds(off, STRIPE)])
    plsc.subcore_barrier()
    for e_ref, deg in ((e_t, deg_t), (e_e, deg_e), (e_p, deg_p)):
        n_chunks = (CHUNKS - wid + NW - 1) // NW

        def body(i, carry, e_ref=e_ref, deg=deg):
            row0 = (wid + NW * i) * K
            pltpu.sync_copy(e_ref.at[1, pl.ds(row0, K)], idx_v)
            descs = [
                pltpu.async_copy(ones_v, deg.at[idx_v.at[j]], sem, add=True)
                for j in range(K)
            ]
            for d in descs:
                d.wait()
            return carry

        lax.fori_loop(0, n_chunks, body, 0)
    plsc.subcore_barrier()
    for deg, out in ((deg_t, out_t), (deg_e, out_e), (deg_p, out_p)):
        pltpu.sync_copy(deg.at[pl.ds(off, STRIPE)], stage1)
        pltpu.sync_copy(stage1, out.at[pl.ds(cid * NPAD + off, STRIPE)])


def _msg_body(e_t, e_e, e_p, g4_t, g4_e, g4_p, dv_t, dv_e, dv_p,
              z4_hbm, z1_hbm, ao_t, ao_e, ao_p, so_t, so_e, so_p,
              agg, s, idx2, rows, dvals, stage4, stage1, sem):
    cid = lax.axis_index("c")
    sid = lax.axis_index("s")
    wid = cid * NS + sid
    off = sid * STRIPE
    for e_ref, g4, dv, ao, so in (
        (e_t, g4_t, dv_t, ao_t, so_t),
        (e_e, g4_e, dv_e, ao_e, so_e),
        (e_p, g4_p, dv_p, ao_p, so_p),
    ):
        pltpu.sync_copy(z4_hbm, stage4)
        pltpu.sync_copy(stage4, agg.at[pl.ds(off, STRIPE), :])
        pltpu.sync_copy(z1_hbm, stage1)
        pltpu.sync_copy(stage1, s.at[pl.ds(off, STRIPE)])
        plsc.subcore_barrier()
        n_chunks = (CHUNKS - wid + NW - 1) // NW

        def body(i, carry, e_ref=e_ref, g4=g4, dv=dv):
            row0 = (wid + NW * i) * K
            pltpu.sync_copy(e_ref.at[:, pl.ds(row0, K)], idx2)
            descs = [
                pltpu.async_copy(g4.at[idx2.at[0, j]], rows.at[j], sem)
                for j in range(K)
            ] + [
                pltpu.async_copy(dv.at[idx2.at[1, j]], dvals.at[j], sem)
                for j in range(K)
            ]
            for d in descs:
                d.wait()
            descs = [
                pltpu.async_copy(rows.at[j], agg.at[idx2.at[1, j]], sem, add=True)
                for j in range(K)
            ] + [
                pltpu.async_copy(dvals.at[j], s.at[idx2.at[0, j]], sem, add=True)
                for j in range(K)
            ]
            for d in descs:
                d.wait()
            return carry

        lax.fori_loop(0, n_chunks, body, 0)
        plsc.subcore_barrier()
        pltpu.sync_copy(agg.at[pl.ds(off, STRIPE), :], stage4)
        pltpu.sync_copy(stage4, ao.at[pl.ds(cid * NPAD + off, STRIPE), :])
        pltpu.sync_copy(s.at[pl.ds(off, STRIPE)], stage1)
        pltpu.sync_copy(stage1, so.at[pl.ds(cid * NPAD + off, STRIPE)])


def _prep_body(degp_ref, x_ref, dinv_ref, g4_ref):
    i = pl.program_id(0)
    dsum = degp_ref[:, 0, :] + degp_ref[:, 1, :] + 1.0
    rowid = lax.broadcasted_iota(jnp.int32, (3, BLK), 1) + i * BLK
    dinv = jnp.where(rowid < N, lax.rsqrt(dsum), 0.0)
    dinv_ref[...] = dinv
    g4_ref[...] = x_ref[...] * dinv[:, :, None]


def _acc_body(aggp_ref, sp_ref, dinv_ref, g4_ref, w1_ref, b1_ref, out_ref):
    i = pl.program_id(0)
    a4 = aggp_ref[:, 0] + aggp_ref[:, 1] + g4_ref[...]
    dinv = dinv_ref[...]
    s2 = sp_ref[:, 0, :] + sp_ref[:, 1, :]
    crow = dinv * (s2 + dinv)
    accs = []
    for g in range(3):
        zw = jnp.dot(a4[g], w1_ref[g], preferred_element_type=jnp.float32)
        dinv_col = jnp.transpose(dinv[g:g + 1, :])
        h = jax.nn.relu(zw * dinv_col + b1_ref[g:g + 1, :])
        accs.append(jnp.dot(crow[g:g + 1, :], h,
                            preferred_element_type=jnp.float32))

    @pl.when(i == 0)
    def _():
        out_ref[...] = jnp.zeros_like(out_ref)

    out_ref[...] += jnp.concatenate(accs, axis=0)


def _head_body(accs_ref, w2_ref, b2_ref, wfc_ref, bfc_ref, out_ref):
    ms = []
    for g in range(3):
        a = accs_ref[g:g + 1, :]
        m = jnp.dot(a, w2_ref[g], preferred_element_type=jnp.float32)
        ms.append(m / float(N) + b2_ref[g:g + 1, :])
    comb = jnp.concatenate(ms, axis=1)
    o = jnp.dot(comb, wfc_ref[...], preferred_element_type=jnp.float32)
    out_ref[...] = jax.nn.sigmoid(o + bfc_ref[...])


def _pad_x(x):
    return jnp.pad(x, ((0, NPAD - N), (0, 0)))


def kernel(target_x, target_edge_index, e3_ligase_x, e3_ligase_edge_index,
           protac_x, protac_edge_index, W1t, b1t, W2t, b2t, W1e, b1e, W2e, b2e,
           W1p, b1p, W2p, b2p, Wfc, bfc):
    f32 = jnp.float32
    e_t = jnp.reshape(target_edge_index.astype(jnp.int32), (2, R, LANES))
    e_e = jnp.reshape(e3_ligase_edge_index.astype(jnp.int32), (2, R, LANES))
    e_p = jnp.reshape(protac_edge_index.astype(jnp.int32), (2, R, LANES))
    xs = jnp.stack([_pad_x(target_x), _pad_x(e3_ligase_x), _pad_x(protac_x)])

    ones_hbm = jnp.ones((LANES,), f32)
    z1_hbm = jnp.zeros((STRIPE,), f32)
    z4_hbm = jnp.zeros((STRIPE, 4), f32)

    mesh = plsc.VectorSubcoreMesh(
        core_axis_name="c", subcore_axis_name="s",
        num_cores=NC, num_subcores=NS)

    degp = pl.kernel(
        _deg_body,
        compiler_params=pltpu.CompilerParams(use_tc_tiling_on_sc=False),
        out_type=[jax.ShapeDtypeStruct((NC * NPAD,), f32)] * 3,
        mesh=mesh,
        scratch_types=[
            pltpu.VMEM_SHARED((NPAD,), f32),
            pltpu.VMEM_SHARED((NPAD,), f32),
            pltpu.VMEM_SHARED((NPAD,), f32),
            pltpu.VMEM((LANES,), f32),
            pltpu.VMEM((K, LANES), jnp.int32),
            pltpu.VMEM((STRIPE,), f32),
            pltpu.SemaphoreType.DMA,
        ],
    )(e_t, e_e, e_p, ones_hbm, z1_hbm)
    degp = jnp.stack(degp).reshape(3, NC, NPAD)

    dinv_all, g4_all = pl.pallas_call(
        _prep_body,
        grid=(NB,),
        in_specs=[
            pl.BlockSpec((3, NC, BLK), lambda i: (0, 0, i)),
            pl.BlockSpec((3, BLK, 4), lambda i: (0, i, 0)),
        ],
        out_specs=[
            pl.BlockSpec((3, BLK), lambda i: (0, i)),
            pl.BlockSpec((3, BLK, 4), lambda i: (0, i, 0)),
        ],
        out_shape=[
            jax.ShapeDtypeStruct((3, NPAD), f32),
            jax.ShapeDtypeStruct((3, NPAD, 4), f32),
        ],
    )(degp, xs)

    msg_outs = pl.kernel(
        _msg_body,
        compiler_params=pltpu.CompilerParams(use_tc_tiling_on_sc=False),
        out_type=[jax.ShapeDtypeStruct((NC * NPAD, 4), f32)] * 3
                 + [jax.ShapeDtypeStruct((NC * NPAD,), f32)] * 3,
        mesh=mesh,
        scratch_types=[
            pltpu.VMEM_SHARED((NPAD, 4), f32),
            pltpu.VMEM_SHARED((NPAD,), f32),
            pltpu.VMEM((2, K, LANES), jnp.int32),
            pltpu.VMEM((K, LANES, 4), f32),
            pltpu.VMEM((K, LANES), f32),
            pltpu.VMEM((STRIPE, 4), f32),
            pltpu.VMEM((STRIPE,), f32),
            pltpu.SemaphoreType.DMA,
        ],
    )(e_t, e_e, e_p,
      g4_all[0], g4_all[1], g4_all[2],
      dinv_all[0], dinv_all[1], dinv_all[2],
      z4_hbm, z1_hbm)
    aggp = jnp.stack(msg_outs[:3]).reshape(3, NC, NPAD, 4)
    sp = jnp.stack(msg_outs[3:]).reshape(3, NC, NPAD)

    w1s = jnp.stack([W1t, W1e, W1p])
    b1s = jnp.stack([b1t, b1e, b1p])
    accs = pl.pallas_call(
        _acc_body,
        grid=(NB,),
        in_specs=[
            pl.BlockSpec((3, NC, BLK, 4), lambda i: (0, 0, i, 0)),
            pl.BlockSpec((3, NC, BLK), lambda i: (0, 0, i)),
            pl.BlockSpec((3, BLK), lambda i: (0, i)),
            pl.BlockSpec((3, BLK, 4), lambda i: (0, i, 0)),
            pl.BlockSpec((3, 4, 64), lambda i: (0, 0, 0)),
            pl.BlockSpec((3, 64), lambda i: (0, 0)),
        ],
        out_specs=pl.BlockSpec((3, 64), lambda i: (0, 0)),
        out_shape=jax.ShapeDtypeStruct((3, 64), f32),
    )(aggp, sp, dinv_all, g4_all, w1s, b1s)

    w2s = jnp.stack([W2t, W2e, W2p])
    b2s = jnp.stack([b2t, b2e, b2p])
    out = pl.pallas_call(
        _head_body,
        out_shape=jax.ShapeDtypeStruct((1, 1), f32),
    )(accs, w2s, b2s, Wfc, bfc.reshape(1, 1))
    return out.reshape(1)


# speedup vs baseline: 106.9355x; 1.1271x over previous
import functools

import jax
import jax.numpy as jnp
from jax import lax
from jax.experimental import pallas as pl
from jax.experimental.pallas import tpu as pltpu
from jax.experimental.pallas import tpu_sc as plsc

N = 100000
E = 3200000
LANES = 128
R = E // LANES
K = 25
CHUNKS = R // K
NC, NS = 2, 16
NW = NC * NS
NPAD = 100352
STRIPE = NPAD // NS
BLK = 2048
NB = NPAD // BLK


def _deg_body(e_t, e_e, e_p, ones_hbm, z1_hbm, out_t, out_e, out_p,
              deg_t, deg_e, deg_p, ones_v, idx_v, stage1, sem):
    cid = lax.axis_index("c")
    sid = lax.axis_index("s")
    wid = cid * NS + sid
    off = sid * STRIPE
    pltpu.sync_copy(ones_hbm, ones_v)
    pltpu.sync_copy(z1_hbm, stage1)
    for deg in (deg_t, deg_e, deg_p):
        pltpu.sync_copy(stage1, deg.at[pl.ds(off, STRIPE)])
    plsc.subcore_barrier()
    for e_ref, deg in ((e_t, deg_t), (e_e, deg_e), (e_p, deg_p)):
        n_chunks = (CHUNKS - wid + NW - 1) // NW

        def body(i, carry, e_ref=e_ref, deg=deg):
            row0 = (wid + NW * i) * K
            pltpu.sync_copy(e_ref.at[1, pl.ds(row0, K)], idx_v)
            descs = [
                pltpu.async_copy(ones_v, deg.at[idx_v.at[j]], sem, add=True)
                for j in range(K)
            ]
            for d in descs:
                d.wait()
            return carry

        lax.fori_loop(0, n_chunks, body, 0)
    plsc.subcore_barrier()
    for deg, out in ((deg_t, out_t), (deg_e, out_e), (deg_p, out_p)):
        pltpu.sync_copy(deg.at[pl.ds(off, STRIPE)], stage1)
        pltpu.sync_copy(stage1, out.at[pl.ds(cid * NPAD + off, STRIPE)])


def _msg_body(e_t, e_e, e_p, g4_t, g4_e, g4_p, dv_t, dv_e, dv_p,
              z4_hbm, z1_hbm, ao_t, ao_e, ao_p, so_t, so_e, so_p,
              agg, s, idx2, rows, dvals, stage4, stage1, sem):
    cid = lax.axis_index("c")
    sid = lax.axis_index("s")
    wid = cid * NS + sid
    off = sid * STRIPE
    for e_ref, g4, dv, ao, so in (
        (e_t, g4_t, dv_t, ao_t, so_t),
        (e_e, g4_e, dv_e, ao_e, so_e),
        (e_p, g4_p, dv_p, ao_p, so_p),
    ):
        pltpu.sync_copy(z4_hbm, stage4)
        pltpu.sync_copy(stage4, agg.at[pl.ds(off, STRIPE), :])
        pltpu.sync_copy(z1_hbm, stage1)
        pltpu.sync_copy(stage1, s.at[pl.ds(off, STRIPE)])
        plsc.subcore_barrier()
        n_chunks = (CHUNKS - wid + NW - 1) // NW

        def body(i, carry, e_ref=e_ref, g4=g4, dv=dv):
            row0 = (wid + NW * i) * K
            pltpu.sync_copy(e_ref.at[:, pl.ds(row0, K)], idx2)
            descs = [
                pltpu.async_copy(g4.at[idx2.at[0, j]], rows.at[j], sem)
                for j in range(K)
            ] + [
                pltpu.async_copy(dv.at[idx2.at[1, j]], dvals.at[j], sem)
                for j in range(K)
            ]
            for d in descs:
                d.wait()
            descs = [
                pltpu.async_copy(rows.at[j], agg.at[idx2.at[1, j]], sem, add=True)
                for j in range(K)
            ] + [
                pltpu.async_copy(dvals.at[j], s.at[idx2.at[0, j]], sem, add=True)
                for j in range(K)
            ]
            for d in descs:
                d.wait()
            return carry

        lax.fori_loop(0, n_chunks, body, 0)
        plsc.subcore_barrier()
        pltpu.sync_copy(agg.at[pl.ds(off, STRIPE), :], stage4)
        pltpu.sync_copy(stage4, ao.at[pl.ds(cid * NPAD + off, STRIPE), :])
        pltpu.sync_copy(s.at[pl.ds(off, STRIPE)], stage1)
        pltpu.sync_copy(stage1, so.at[pl.ds(cid * NPAD + off, STRIPE)])


def _prep_body(dp_t, dp_e, dp_p, x_t, x_e, x_p,
               dv_t, dv_e, dv_p, g4_t, g4_e, g4_p):
    i = pl.program_id(0)
    rowid = lax.broadcasted_iota(jnp.int32, (1, BLK), 1) + i * BLK
    mask = rowid < N
    for dp, x, dv, g4 in ((dp_t, x_t, dv_t, g4_t), (dp_e, x_e, dv_e, g4_e),
                          (dp_p, x_p, dv_p, g4_p)):
        dsum = dp[0:1, :] + dp[1:2, :] + 1.0
        dinv = jnp.where(mask, lax.rsqrt(dsum), 0.0)
        dv[...] = jnp.reshape(dinv, (BLK,))
        dcol = jnp.transpose(dinv)
        g4[...] = jnp.where(dcol > 0.0, x[...] * dcol, 0.0)


def _acc_body(ap_t, ap_e, ap_p, sp_t, sp_e, sp_p, dv_t, dv_e, dv_p,
              g4_t, g4_e, g4_p, w1_t, w1_e, w1_p, b1_t, b1_e, b1_p,
              w2_t, w2_e, w2_p, b2_t, b2_e, b2_p, wfc, bfc, out_ref, acc_ref):
    i = pl.program_id(0)

    @pl.when(i == 0)
    def _():
        acc_ref[...] = jnp.zeros_like(acc_ref)

    for g, (ap, sp, dv, g4, w1, b1) in enumerate((
        (ap_t, sp_t, dv_t, g4_t, w1_t, b1_t),
        (ap_e, sp_e, dv_e, g4_e, w1_e, b1_e),
        (ap_p, sp_p, dv_p, g4_p, w1_p, b1_p),
    )):
        z4 = ap[0] + ap[1] + g4[...]
        zw = jnp.dot(z4, w1[...], preferred_element_type=jnp.float32)
        drow = jnp.reshape(dv[...], (1, BLK))
        dcol = jnp.transpose(drow)
        h = jax.nn.relu(zw * dcol + b1[...])
        srow = sp[0:1, :] + sp[1:2, :]
        crow = drow * (srow + drow)
        acc_ref[g:g + 1, :] += jnp.dot(crow, h,
                                       preferred_element_type=jnp.float32)

    @pl.when(i == NB - 1)
    def _():
        ms = []
        for g, (w2, b2) in enumerate(((w2_t, b2_t), (w2_e, b2_e),
                                      (w2_p, b2_p))):
            m = jnp.dot(acc_ref[g:g + 1, :], w2[...],
                        preferred_element_type=jnp.float32)
            ms.append(m / float(N) + b2[...])
        comb = jnp.concatenate(ms, axis=1)
        o = jnp.dot(comb, wfc[...], preferred_element_type=jnp.float32)
        out_ref[...] = jax.nn.sigmoid(o + bfc[...])


def _pad_x(x):
    return jnp.pad(x, ((0, NPAD - N), (0, 0)))


def kernel(target_x, target_edge_index, e3_ligase_x, e3_ligase_edge_index,
           protac_x, protac_edge_index, W1t, b1t, W2t, b2t, W1e, b1e, W2e, b2e,
           W1p, b1p, W2p, b2p, Wfc, bfc):
    f32 = jnp.float32
    e_t = jnp.reshape(target_edge_index.astype(jnp.int32), (2, R, LANES))
    e_e = jnp.reshape(e3_ligase_edge_index.astype(jnp.int32), (2, R, LANES))
    e_p = jnp.reshape(protac_edge_index.astype(jnp.int32), (2, R, LANES))

    ones_hbm = jnp.ones((LANES,), f32)
    z1_hbm = jnp.zeros((STRIPE,), f32)
    z4_hbm = jnp.zeros((STRIPE, 4), f32)

    mesh = plsc.VectorSubcoreMesh(
        core_axis_name="c", subcore_axis_name="s",
        num_cores=NC, num_subcores=NS)

    degp = pl.kernel(
        _deg_body,
        compiler_params=pltpu.CompilerParams(use_tc_tiling_on_sc=False),
        out_type=[jax.ShapeDtypeStruct((NC * NPAD,), f32)] * 3,
        mesh=mesh,
        scratch_types=[
            pltpu.VMEM_SHARED((NPAD,), f32),
            pltpu.VMEM_SHARED((NPAD,), f32),
            pltpu.VMEM_SHARED((NPAD,), f32),
            pltpu.VMEM((LANES,), f32),
            pltpu.VMEM((K, LANES), jnp.int32),
            pltpu.VMEM((STRIPE,), f32),
            pltpu.SemaphoreType.DMA,
        ],
    )(e_t, e_e, e_p, ones_hbm, z1_hbm)
    degp = [d.reshape(NC, NPAD) for d in degp]

    prep_outs = pl.pallas_call(
        _prep_body,
        grid=(NB,),
        in_specs=[pl.BlockSpec((NC, BLK), lambda i: (0, i))] * 3
                 + [pl.BlockSpec((BLK, 4), lambda i: (i, 0))] * 3,
        out_specs=[pl.BlockSpec((BLK,), lambda i: (i,))] * 3
                  + [pl.BlockSpec((BLK, 4), lambda i: (i, 0))] * 3,
        out_shape=[jax.ShapeDtypeStruct((NPAD,), f32)] * 3
                  + [jax.ShapeDtypeStruct((NPAD, 4), f32)] * 3,
    )(*degp, target_x, e3_ligase_x, protac_x)
    dv_t, dv_e, dv_p, g4_t, g4_e, g4_p = prep_outs

    msg_outs = pl.kernel(
        _msg_body,
        compiler_params=pltpu.CompilerParams(use_tc_tiling_on_sc=False),
        out_type=[jax.ShapeDtypeStruct((NC * NPAD, 4), f32)] * 3
                 + [jax.ShapeDtypeStruct((NC * NPAD,), f32)] * 3,
        mesh=mesh,
        scratch_types=[
            pltpu.VMEM_SHARED((NPAD, 4), f32),
            pltpu.VMEM_SHARED((NPAD,), f32),
            pltpu.VMEM((2, K, LANES), jnp.int32),
            pltpu.VMEM((K, LANES, 4), f32),
            pltpu.VMEM((K, LANES), f32),
            pltpu.VMEM((STRIPE, 4), f32),
            pltpu.VMEM((STRIPE,), f32),
            pltpu.SemaphoreType.DMA,
        ],
    )(e_t, e_e, e_p, g4_t, g4_e, g4_p, dv_t, dv_e, dv_p, z4_hbm, z1_hbm)
    aggp = [a.reshape(NC, NPAD, 4) for a in msg_outs[:3]]
    sp = [s.reshape(NC, NPAD) for s in msg_outs[3:]]

    full = lambda s: pl.BlockSpec(s, lambda i: tuple(0 for _ in s))
    out = pl.pallas_call(
        _acc_body,
        grid=(NB,),
        in_specs=[pl.BlockSpec((NC, BLK, 4), lambda i: (0, i, 0))] * 3
                 + [pl.BlockSpec((NC, BLK), lambda i: (0, i))] * 3
                 + [pl.BlockSpec((BLK,), lambda i: (i,))] * 3
                 + [pl.BlockSpec((BLK, 4), lambda i: (i, 0))] * 3
                 + [full((4, 64))] * 3 + [full((1, 64))] * 3
                 + [full((64, 32))] * 3 + [full((1, 32))] * 3
                 + [full((96, 1)), full((1, 1))],
        out_specs=pl.BlockSpec((1, 1), lambda i: (0, 0)),
        out_shape=jax.ShapeDtypeStruct((1, 1), f32),
        scratch_shapes=[pltpu.VMEM((3, 64), f32)],
    )(*aggp, *sp, dv_t, dv_e, dv_p, g4_t, g4_e, g4_p,
      W1t, W1e, W1p, b1t.reshape(1, 64), b1e.reshape(1, 64), b1p.reshape(1, 64),
      W2t, W2e, W2p, b2t.reshape(1, 32), b2e.reshape(1, 32), b2p.reshape(1, 32),
      Wfc, bfc.reshape(1, 1))
    return out.reshape(1)


# speedup vs baseline: 121.0765x; 1.1322x over previous
import jax
import jax.numpy as jnp
from jax import lax
from jax.experimental import pallas as pl
from jax.experimental.pallas import tpu as pltpu
from jax.experimental.pallas import tpu_sc as plsc

N = 100000
E = 3200000
LANES = 128
R = E // LANES
K = 25
CHUNKS = R // K
NC, NS = 2, 16
NW = NC * NS
NPAD = 100352
STRIPE = NPAD // NS
BLK = 2048
NB = NPAD // BLK


def _deg_body(e, ones_hbm, z1_hbm, out, deg, ones_v, idx_v, stage1, sem):
    cid = lax.axis_index("c")
    sid = lax.axis_index("s")
    wid = cid * NS + sid
    off = sid * STRIPE
    pltpu.sync_copy(ones_hbm, ones_v)
    pltpu.sync_copy(z1_hbm, stage1)
    pltpu.sync_copy(stage1, deg.at[pl.ds(off, STRIPE)])
    plsc.subcore_barrier()
    n_chunks = (CHUNKS - wid + NW - 1) // NW

    def body(i, carry):
        row0 = (wid + NW * i) * K
        pltpu.sync_copy(e.at[1, pl.ds(row0, K)], idx_v)
        descs = [
            pltpu.async_copy(ones_v, deg.at[idx_v.at[j]], sem, add=True)
            for j in range(K)
        ]
        for d in descs:
            d.wait()
        return carry

    lax.fori_loop(0, n_chunks, body, 0)
    plsc.subcore_barrier()
    pltpu.sync_copy(deg.at[pl.ds(off, STRIPE)], stage1)
    pltpu.sync_copy(stage1, out.at[pl.ds(cid * NPAD + off, STRIPE)])


def _msg_body(e, g4, dv, z4_hbm, z1_hbm, ao, so,
              agg, s, idx2, rows, dvals, stage4, stage1, sem):
    cid = lax.axis_index("c")
    sid = lax.axis_index("s")
    wid = cid * NS + sid
    off = sid * STRIPE
    pltpu.sync_copy(z4_hbm, stage4)
    pltpu.sync_copy(stage4, agg.at[pl.ds(off, STRIPE), :])
    pltpu.sync_copy(z1_hbm, stage1)
    pltpu.sync_copy(stage1, s.at[pl.ds(off, STRIPE)])
    plsc.subcore_barrier()
    n_chunks = (CHUNKS - wid + NW - 1) // NW

    def body(i, carry):
        row0 = (wid + NW * i) * K
        pltpu.sync_copy(e.at[:, pl.ds(row0, K)], idx2)
        descs = [
            pltpu.async_copy(g4.at[idx2.at[0, j]], rows.at[j], sem)
            for j in range(K)
        ] + [
            pltpu.async_copy(dv.at[idx2.at[1, j]], dvals.at[j], sem)
            for j in range(K)
        ]
        for d in descs:
            d.wait()
        descs = [
            pltpu.async_copy(rows.at[j], agg.at[idx2.at[1, j]], sem, add=True)
            for j in range(K)
        ] + [
            pltpu.async_copy(dvals.at[j], s.at[idx2.at[0, j]], sem, add=True)
            for j in range(K)
        ]
        for d in descs:
            d.wait()
        return carry

    lax.fori_loop(0, n_chunks, body, 0)
    plsc.subcore_barrier()
    pltpu.sync_copy(agg.at[pl.ds(off, STRIPE), :], stage4)
    pltpu.sync_copy(stage4, ao.at[pl.ds(cid * NPAD + off, STRIPE), :])
    pltpu.sync_copy(s.at[pl.ds(off, STRIPE)], stage1)
    pltpu.sync_copy(stage1, so.at[pl.ds(cid * NPAD + off, STRIPE)])


def _prep_body(dp, x, dv, g4):
    i = pl.program_id(0)
    rowid = lax.broadcasted_iota(jnp.int32, (1, BLK), 1) + i * BLK
    mask = rowid < N
    dsum = dp[0:1, :] + dp[1:2, :] + 1.0
    dinv = jnp.where(mask, lax.rsqrt(dsum), 0.0)
    dv[...] = jnp.reshape(dinv, (BLK,))
    dcol = jnp.transpose(dinv)
    g4[...] = jnp.where(dcol > 0.0, x[...] * dcol, 0.0)


def _acc_body(ap, sp, dv, g4, w1, b1, out_ref):
    i = pl.program_id(0)

    @pl.when(i == 0)
    def _():
        out_ref[...] = jnp.zeros_like(out_ref)

    z4 = ap[0] + ap[1] + g4[...]
    zw = jnp.dot(z4, w1[...], preferred_element_type=jnp.float32)
    drow = jnp.reshape(dv[...], (1, BLK))
    dcol = jnp.transpose(drow)
    h = jax.nn.relu(zw * dcol + b1[...])
    srow = sp[0:1, :] + sp[1:2, :]
    crow = drow * (srow + drow)
    out_ref[...] += jnp.dot(crow, h, preferred_element_type=jnp.float32)


def _head_body(m_t, m_e, m_p, w2_t, w2_e, w2_p, b2_t, b2_e, b2_p,
               wfc, bfc, out_ref):
    ms = []
    for m, w2, b2 in ((m_t, w2_t, b2_t), (m_e, w2_e, b2_e), (m_p, w2_p, b2_p)):
        o = jnp.dot(m[...], w2[...], preferred_element_type=jnp.float32)
        ms.append(o / float(N) + b2[...])
    comb = jnp.concatenate(ms, axis=1)
    o = jnp.dot(comb, wfc[...], preferred_element_type=jnp.float32)
    out_ref[...] = jax.nn.sigmoid(o + bfc[...])


def kernel(target_x, target_edge_index, e3_ligase_x, e3_ligase_edge_index,
           protac_x, protac_edge_index, W1t, b1t, W2t, b2t, W1e, b1e, W2e, b2e,
           W1p, b1p, W2p, b2p, Wfc, bfc):
    f32 = jnp.float32
    ones_hbm = jnp.ones((LANES,), f32)
    z1_hbm = jnp.zeros((STRIPE,), f32)
    z4_hbm = jnp.zeros((STRIPE, 4), f32)

    mesh = plsc.VectorSubcoreMesh(
        core_axis_name="c", subcore_axis_name="s",
        num_cores=NC, num_subcores=NS)

    deg_call = pl.kernel(
        _deg_body,
        compiler_params=pltpu.CompilerParams(use_tc_tiling_on_sc=False),
        out_type=jax.ShapeDtypeStruct((NC * NPAD,), f32),
        mesh=mesh,
        scratch_types=[
            pltpu.VMEM_SHARED((NPAD,), f32),
            pltpu.VMEM((LANES,), f32),
            pltpu.VMEM((K, LANES), jnp.int32),
            pltpu.VMEM((STRIPE,), f32),
            pltpu.SemaphoreType.DMA,
        ],
    )

    prep_call = pl.pallas_call(
        _prep_body,
        grid=(NB,),
        in_specs=[pl.BlockSpec((NC, BLK), lambda i: (0, i)),
                  pl.BlockSpec((BLK, 4), lambda i: (i, 0))],
        out_specs=[pl.BlockSpec((BLK,), lambda i: (i,)),
                   pl.BlockSpec((BLK, 4), lambda i: (i, 0))],
        out_shape=[jax.ShapeDtypeStruct((NPAD,), f32),
                   jax.ShapeDtypeStruct((NPAD, 4), f32)],
    )

    msg_call = pl.kernel(
        _msg_body,
        compiler_params=pltpu.CompilerParams(use_tc_tiling_on_sc=False),
        out_type=[jax.ShapeDtypeStruct((NC * NPAD, 4), f32),
                  jax.ShapeDtypeStruct((NC * NPAD,), f32)],
        mesh=mesh,
        scratch_types=[
            pltpu.VMEM_SHARED((NPAD, 4), f32),
            pltpu.VMEM_SHARED((NPAD,), f32),
            pltpu.VMEM((2, K, LANES), jnp.int32),
            pltpu.VMEM((K, LANES, 4), f32),
            pltpu.VMEM((K, LANES), f32),
            pltpu.VMEM((STRIPE, 4), f32),
            pltpu.VMEM((STRIPE,), f32),
            pltpu.SemaphoreType.DMA,
        ],
    )

    full = lambda s: pl.BlockSpec(s, lambda i: tuple(0 for _ in s))
    acc_call = pl.pallas_call(
        _acc_body,
        grid=(NB,),
        in_specs=[pl.BlockSpec((NC, BLK, 4), lambda i: (0, i, 0)),
                  pl.BlockSpec((NC, BLK), lambda i: (0, i)),
                  pl.BlockSpec((BLK,), lambda i: (i,)),
                  pl.BlockSpec((BLK, 4), lambda i: (i, 0)),
                  full((4, 64)), full((1, 64))],
        out_specs=pl.BlockSpec((1, 64), lambda i: (0, 0)),
        out_shape=jax.ShapeDtypeStruct((1, 64), f32),
    )

    graphs = (
        (target_x, target_edge_index, W1t, b1t),
        (e3_ligase_x, e3_ligase_edge_index, W1e, b1e),
        (protac_x, protac_edge_index, W1p, b1p),
    )
    msums = []
    for x, edge_index, w1, b1 in graphs:
        e = jnp.reshape(edge_index.astype(jnp.int32), (2, R, LANES))
        degp = deg_call(e, ones_hbm, z1_hbm).reshape(NC, NPAD)
        dv, g4 = prep_call(degp, x)
        ao, so = msg_call(e, g4, dv, z4_hbm, z1_hbm)
        m = acc_call(ao.reshape(NC, NPAD, 4), so.reshape(NC, NPAD),
                     dv, g4, w1, b1.reshape(1, 64))
        msums.append(m)

    full0 = lambda s: pl.BlockSpec(s, lambda: tuple(0 for _ in s))
    out = pl.pallas_call(
        _head_body,
        in_specs=[full0((1, 64))] * 3 + [full0((64, 32))] * 3
                 + [full0((1, 32))] * 3 + [full0((96, 1)), full0((1, 1))],
        out_specs=full0((1, 1)),
        out_shape=jax.ShapeDtypeStruct((1, 1), f32),
    )(*msums, W2t, W2e, W2p, b2t.reshape(1, 32), b2e.reshape(1, 32),
      b2p.reshape(1, 32), Wfc, bfc.reshape(1, 1))
    return out.reshape(1)
